# Initial kernel scaffold; baseline (speedup 1.0000x reference)
import jax
import jax.numpy as jnp
from jax.experimental import pallas as pl


def kernel(x, y):
    raise NotImplementedError("write your pallas kernel here")


# speedup vs baseline: 30324.6868x; 30324.6868x over previous
import functools

import jax
import jax.numpy as jnp
from jax import lax
from jax.experimental import pallas as pl
from jax.experimental.pallas import tpu as pltpu
from jax.experimental.pallas import tpu_sc as plsc

B, S, T, SP = 2, 8, 2048, 128
DT = 0.001
L = 16
NC, NS = 2, 16
NW = NC * NS
LG = (B * S * SP) // L
LG_PER_W = LG // NW


def _min_tc_body(x_ref, y_ref, o_ref):
    m = jnp.minimum(jnp.min(x_ref[...]), jnp.min(y_ref[...]))

    @pl.when(pl.program_id(0) == 0)
    def _():
        o_ref[0, 0] = m

    @pl.when(pl.program_id(0) > 0)
    def _():
        o_ref[0, 0] = jnp.minimum(o_ref[0, 0], m)


def _global_min(x, y):
    x2 = x.reshape(4096, 1024)
    y2 = y.reshape(4096, 1024)
    grid = 8
    blk = 4096 // grid
    return pl.pallas_call(
        _min_tc_body,
        grid=(grid,),
        in_specs=[
            pl.BlockSpec((blk, 1024), lambda i: (i, 0)),
            pl.BlockSpec((blk, 1024), lambda i: (i, 0)),
        ],
        out_specs=pl.BlockSpec(memory_space=pltpu.SMEM),
        out_shape=jax.ShapeDtypeStruct((1, 1), jnp.float32),
    )(x2, y2)


def _sc_body(x_hbm, y_hbm, min_hbm, out_hbm, xs, ys, minv, outv):
    wid = lax.axis_index("s") * NC + lax.axis_index("c")
    pltpu.sync_copy(min_hbm.at[:], minv)
    m = minv[...]
    shift = jnp.where(m < 0.0, 1.1 * m, jnp.zeros_like(m))
    lane = lax.iota(jnp.int32, L)
    zf = jnp.zeros((L,), jnp.float32)
    zi = jnp.zeros((L,), jnp.int32)
    big = jnp.full((L,), 3.0e38, jnp.float32)

    acc_out = zf
    for k in range(LG_PER_W):
        g = wid * LG_PER_W + k
        b = g // (LG // B)
        rem = g % (LG // B)
        sidx = rem // (SP // L)
        sp0 = (rem % (SP // L)) * L
        pltpu.sync_copy(x_hbm.at[b, sidx, :, pl.ds(sp0, L)], xs)
        pltpu.sync_copy(y_hbm.at[b, sidx, :, pl.ds(sp0, L)], ys)

        def sum_body(i, carry):
            ax, ay = carry
            return ax + xs[i], ay + ys[i]

        ax, ay = lax.fori_loop(0, T, sum_body, (zf, zf), unroll=8)
        sx = ax - jnp.float32(T) * shift
        sy = ay - jnp.float32(T) * shift
        keep = jnp.logical_and(sx != 0.0, sy != 0.0)
        invdx = 1.0 / (sx + 1e-10)
        invdy = 1.0 / (sy + 1e-10)

        def cs_body(i, carry):
            cx, cy = carry
            cx = cx + (xs[i] - shift) * invdx
            cy = cy + (ys[i] - shift) * invdy
            xs[i] = cx
            ys[i] = cy
            return cx, cy

        lax.fori_loop(0, T, cs_body, (zf, zf), unroll=8)

        def mg_body(step, carry):
            i, j, prev, acc = carry
            iu = jnp.minimum(i, T - 1)
            ju = jnp.minimum(j, T - 1)
            uc = plsc.load_gather(xs, [iu, lane])
            vc = plsc.load_gather(ys, [ju, lane])
            uc = jnp.where(i >= T, big, uc)
            vc = jnp.where(j >= T, big, vc)
            take_u = uc <= vc
            q = jnp.minimum(uc, vc)
            td = jnp.float32(DT) * (iu - ju).astype(jnp.float32)
            acc = acc + (q - prev) * (td * td)
            one = jnp.ones((L,), jnp.int32)
            i = i + jnp.where(take_u, one, zi)
            j = j + jnp.where(take_u, zi, one)
            return i, j, q, acc

        _, _, _, w = lax.fori_loop(0, 2 * T, mg_body, (zi, zi, zf, zf),
                                   unroll=4)
        acc_out = acc_out + jnp.where(keep, w, zf)

    outv[...] = acc_out
    pltpu.sync_copy(outv, out_hbm.at[wid])


def _sc_wasserstein(x, y, minvec):
    mesh = plsc.VectorSubcoreMesh(
        core_axis_name="c", subcore_axis_name="s",
        num_cores=NC, num_subcores=NS)
    f = functools.partial(
        pl.kernel,
        out_type=jax.ShapeDtypeStruct((NW, L), jnp.float32),
        mesh=mesh,
        scratch_types=[
            pltpu.VMEM((T, L), jnp.float32),
            pltpu.VMEM((T, L), jnp.float32),
            pltpu.VMEM((L,), jnp.float32),
            pltpu.VMEM((L,), jnp.float32),
        ],
        compiler_params=pltpu.CompilerParams(
            use_tc_tiling_on_sc=False, needs_layout_passes=False),
    )(_sc_body)
    return f(x, y, minvec)


def kernel(x, y):
    mn = _global_min(x, y)
    minvec = jnp.broadcast_to(mn.reshape(1), (L,))
    part = _sc_wasserstein(x, y, minvec)
    return part.reshape(B, NW // B * L).sum(axis=1)


# speedup vs baseline: 42218.3981x; 1.3922x over previous
import functools

import jax
import jax.numpy as jnp
from jax import lax
from jax.experimental import pallas as pl
from jax.experimental.pallas import tpu as pltpu
from jax.experimental.pallas import tpu_sc as plsc

B, S, T, SP = 2, 8, 2048, 128
DT = 0.001
L = 16
NC, NS = 2, 16
NW = NC * NS
LG = (B * S * SP) // L
LG_PER_W = LG // NW
CH = 4


def _min_tc_body(x_ref, y_ref, o_ref):
    m = jnp.minimum(jnp.min(x_ref[...]), jnp.min(y_ref[...]))

    @pl.when(pl.program_id(0) == 0)
    def _():
        o_ref[0, 0] = m

    @pl.when(pl.program_id(0) > 0)
    def _():
        o_ref[0, 0] = jnp.minimum(o_ref[0, 0], m)


def _global_min(x, y):
    x2 = x.reshape(4096, 1024)
    y2 = y.reshape(4096, 1024)
    grid = 8
    blk = 4096 // grid
    return pl.pallas_call(
        _min_tc_body,
        grid=(grid,),
        in_specs=[
            pl.BlockSpec((blk, 1024), lambda i: (i, 0)),
            pl.BlockSpec((blk, 1024), lambda i: (i, 0)),
        ],
        out_specs=pl.BlockSpec(memory_space=pltpu.SMEM),
        out_shape=jax.ShapeDtypeStruct((1, 1), jnp.float32),
    )(x2, y2)


def _sc_body(x_hbm, y_hbm, min_hbm, out_hbm, xs, ys, minv, outv):
    wid = lax.axis_index("s") * NC + lax.axis_index("c")
    pltpu.sync_copy(min_hbm.at[:], minv)
    m = minv[...]
    shift = jnp.where(m < 0.0, 1.1 * m, jnp.zeros_like(m))
    lane = lax.iota(jnp.int32, L)
    zf = jnp.zeros((L,), jnp.float32)
    zi = jnp.zeros((L,), jnp.int32)
    big = jnp.full((L,), 3.0e38, jnp.float32)

    acc_out = zf
    for k in range(LG_PER_W):
        g = wid * LG_PER_W + k
        b = g // (LG // B)
        rem = g % (LG // B)
        sidx = rem // (SP // L)
        sp0 = (rem % (SP // L)) * L
        pltpu.sync_copy(x_hbm.at[b, sidx, :, pl.ds(sp0, L)], xs)
        pltpu.sync_copy(y_hbm.at[b, sidx, :, pl.ds(sp0, L)], ys)

        def sum_body(i, carry):
            ax, ay = carry
            return ax + xs[i], ay + ys[i]

        ax, ay = lax.fori_loop(0, T, sum_body, (zf, zf), unroll=8)
        sx = ax - jnp.float32(T) * shift
        sy = ay - jnp.float32(T) * shift
        keep = jnp.logical_and(sx != 0.0, sy != 0.0)
        invdx = 1.0 / (sx + 1e-10)
        invdy = 1.0 / (sy + 1e-10)

        def cs_body(i, carry):
            cx, cy = carry
            cx = cx + (xs[i] - shift) * invdx
            cy = cy + (ys[i] - shift) * invdy
            xs[i] = cx
            ys[i] = cy
            return cx, cy

        lax.fori_loop(0, T, cs_body, (zf, zf), unroll=8)

        st0 = []
        for c in range(CH):
            k0 = c * (2 * T // CH)
            if k0 == 0:
                st0.append((zi, zi, zf, zf))
                continue
            lo = jnp.full((L,), max(0, k0 - T), jnp.int32)
            hi = jnp.full((L,), min(k0, T), jnp.int32)

            def bs_body(r, carry, k0=k0):
                lo, hi = carry
                mid = lax.shift_right_logical(lo + hi, 1)
                um = plsc.load_gather(xs, [jnp.minimum(mid, T - 1), lane])
                vm = plsc.load_gather(
                    ys, [jnp.maximum(k0 - mid - 1, 0), lane])
                f = um > vm
                active = lo < hi
                hi = jnp.where(jnp.logical_and(active, f), mid, hi)
                lo = jnp.where(jnp.logical_and(active, jnp.logical_not(f)),
                               mid + 1, lo)
                return lo, hi

            i0, _ = lax.fori_loop(0, 12, bs_body, (lo, hi))
            j0 = k0 - i0
            pu = plsc.load_gather(xs, [jnp.maximum(i0 - 1, 0), lane])
            pv = plsc.load_gather(ys, [jnp.maximum(j0 - 1, 0), lane])
            pu = jnp.where(i0 > 0, pu, -big)
            pv = jnp.where(j0 > 0, pv, -big)
            st0.append((i0, j0, jnp.maximum(pu, pv), zf))

        def mg_body(step, carry):
            out = []
            for c in range(CH):
                i, j, prev, acc = carry[c]
                iu = jnp.minimum(i, T - 1)
                ju = jnp.minimum(j, T - 1)
                uc = plsc.load_gather(xs, [iu, lane])
                vc = plsc.load_gather(ys, [ju, lane])
                uc = jnp.where(i >= T, big, uc)
                vc = jnp.where(j >= T, big, vc)
                take_u = uc <= vc
                q = jnp.minimum(uc, vc)
                td = jnp.float32(DT) * (iu - ju).astype(jnp.float32)
                acc = acc + (q - prev) * (td * td)
                ti = take_u.astype(jnp.int32)
                out.append((i + ti, j + (1 - ti), q, acc))
            return tuple(out)

        stf = lax.fori_loop(0, 2 * T // CH, mg_body, tuple(st0), unroll=2)
        w = stf[0][3]
        for c in range(1, CH):
            w = w + stf[c][3]
        acc_out = acc_out + jnp.where(keep, w, zf)

    outv[...] = acc_out
    pltpu.sync_copy(outv, out_hbm.at[wid])


def _sc_wasserstein(x, y, minvec):
    mesh = plsc.VectorSubcoreMesh(
        core_axis_name="c", subcore_axis_name="s",
        num_cores=NC, num_subcores=NS)
    f = functools.partial(
        pl.kernel,
        out_type=jax.ShapeDtypeStruct((NW, L), jnp.float32),
        mesh=mesh,
        scratch_types=[
            pltpu.VMEM((T, L), jnp.float32),
            pltpu.VMEM((T, L), jnp.float32),
            pltpu.VMEM((L,), jnp.float32),
            pltpu.VMEM((L,), jnp.float32),
        ],
        compiler_params=pltpu.CompilerParams(
            use_tc_tiling_on_sc=False, needs_layout_passes=False),
    )(_sc_body)
    return f(x, y, minvec)


def kernel(x, y):
    mn = _global_min(x, y)
    minvec = jnp.broadcast_to(mn.reshape(1), (L,))
    part = _sc_wasserstein(x, y, minvec)
    return part.reshape(B, NW // B * L).sum(axis=1)
